# baseline (speedup 1.0000x reference)
import functools

import jax
import jax.numpy as jnp
from jax import lax
from jax.experimental import pallas as pl
from jax.experimental.pallas import tpu as pltpu
from jax.experimental.pallas import tpu_sc as plsc

N_NODES = 10000
N_EDGES = 320000
D = 128

NC = 2
NS = 16
NW = NC * NS
LANES = 16

CHUNK = 128
NSPLIT = 4
E_SPLIT = N_EDGES // NSPLIT


def _leaky(x):
    return jnp.where(x > 0, x, 0.2 * x)


def _k1_body(h_ref, w1a_ref, w1b_ref, be1_ref, a_ref, b_ref):
    hb = h_ref[...]
    a_ref[...] = (
        jnp.dot(hb, w1a_ref[...], preferred_element_type=jnp.float32)
        + be1_ref[...]
    )
    b_ref[...] = jnp.dot(hb, w1b_ref[...], preferred_element_type=jnp.float32)


def _k1(h, w1a_t, w1b_t, be1):
    blk = 2000
    grid = (N_NODES // blk,)
    return pl.pallas_call(
        _k1_body,
        grid=grid,
        in_specs=[
            pl.BlockSpec((blk, D), lambda i: (i, 0)),
            pl.BlockSpec((D, D), lambda i: (0, 0)),
            pl.BlockSpec((D, D), lambda i: (0, 0)),
            pl.BlockSpec((1, D), lambda i: (0, 0)),
        ],
        out_specs=[
            pl.BlockSpec((blk, D), lambda i: (i, 0)),
            pl.BlockSpec((blk, D), lambda i: (i, 0)),
        ],
        out_shape=[
            jax.ShapeDtypeStruct((N_NODES, D), jnp.float32),
            jax.ShapeDtypeStruct((N_NODES, D), jnp.float32),
        ],
    )(h, w1a_t, w1b_t, be1)


K2CH = 128


def _k2_body(n_edges, a_hbm, b_hbm, row_hbm, col_hbm, cx_hbm, cy_hbm, cz_hbm,
             w1r_hbm, pre_hbm,
             cx, cy, cz, w1r_v, idx_r, idx_c, arow, bcol, obuf, radial,
             sem_i0, sem_i1, sem_g0, sem_g1, sem_out):
    NCHUNK = n_edges // K2CH
    NI = (NCHUNK + NW - 1) // NW // 2 + 1
    cid = lax.axis_index("c")
    sid = lax.axis_index("s")
    wid = sid * NC + cid

    pltpu.sync_copy(cx_hbm, cx)
    pltpu.sync_copy(cy_hbm, cy)
    pltpu.sync_copy(cz_hbm, cz)
    pltpu.sync_copy(w1r_hbm, w1r_v)
    w1rj = [w1r_v[pl.ds(j * LANES, LANES)] for j in range(D // LANES)]

    nq = (NCHUNK - wid + NW - 1) // NW
    sem_i = (sem_i0, sem_i1)
    sem_g = (sem_g0, sem_g1)

    def base_of(j):
        return (wid + jnp.minimum(j, nq - 1) * NW) * K2CH

    def issue_idx(j, p):
        pltpu.async_copy(row_hbm.at[pl.ds(base_of(j), K2CH)],
                         idx_r.at[p], sem_i[p])
        pltpu.async_copy(col_hbm.at[pl.ds(base_of(j), K2CH)],
                         idx_c.at[p], sem_i[p])

    def wait_idx(p):
        pltpu.make_async_copy(row_hbm.at[pl.ds(0, K2CH)],
                              idx_r.at[p], sem_i[p]).wait()
        pltpu.make_async_copy(col_hbm.at[pl.ds(0, K2CH)],
                              idx_c.at[p], sem_i[p]).wait()

    def issue_gat(p):
        pltpu.async_copy(a_hbm.at[idx_r.at[p]], arow.at[p], sem_g[p])
        pltpu.async_copy(b_hbm.at[idx_c.at[p]], bcol.at[p], sem_g[p])

    def wait_gat(p):
        pltpu.make_async_copy(a_hbm.at[idx_r.at[p]], arow.at[p],
                              sem_g[p]).wait()
        pltpu.make_async_copy(b_hbm.at[idx_c.at[p]], bcol.at[p],
                              sem_g[p]).wait()

    def wait_out():
        pltpu.make_async_copy(obuf, pre_hbm.at[pl.ds(0, K2CH)],
                              sem_out).wait()

    issue_idx(0, 0)
    issue_idx(1, 1)
    wait_idx(0)
    issue_gat(0)

    def step(j, p):
        live = j < nq
        wait_gat(p)

        @pl.when(live)
        def _():
            for g in range(K2CH // LANES):
                r16 = idx_r[p, pl.ds(g * LANES, LANES)]
                c16 = idx_c[p, pl.ds(g * LANES, LANES)]
                dx = plsc.load_gather(cx, [r16]) - plsc.load_gather(cx, [c16])
                dy = plsc.load_gather(cy, [r16]) - plsc.load_gather(cy, [c16])
                dz = plsc.load_gather(cz, [r16]) - plsc.load_gather(cz, [c16])
                radial[pl.ds(g * LANES, LANES)] = dx * dx + dy * dy + dz * dz

        issue_idx(j + 2, p)
        wait_idx(1 - p)
        issue_gat(1 - p)

        @pl.when(jnp.logical_and(j > 0, live))
        def _():
            wait_out()

        @pl.when(live)
        def _():
            arow_p = arow.at[p]
            bcol_p = bcol.at[p]

            def edge_body(e, carry2):
                rb = plsc.load_gather(
                    radial, [jnp.full((LANES,), e, jnp.int32)])
                for j2 in range(D // LANES):
                    sl = pl.ds(j2 * LANES, LANES)
                    obuf[e, sl] = arow_p[e, sl] + bcol_p[e, sl] + rb * w1rj[j2]
                return carry2

            lax.fori_loop(0, K2CH, edge_body, 0, unroll=4)
            pltpu.async_copy(obuf, pre_hbm.at[pl.ds(base_of(j), K2CH)],
                             sem_out)

    def loop_body(i, carry):
        step(2 * i, 0)
        step(2 * i + 1, 1)
        return carry

    lax.fori_loop(0, NI, loop_body, 0)

    wait_out()
    wait_gat(0)
    wait_idx(1)


def _k2(a, b, row1, col1, cx_a, cy_a, cz_a, w1r):
    n_edges = row1.shape[0]
    mesh = plsc.VectorSubcoreMesh(core_axis_name="c", subcore_axis_name="s")
    f = pl.kernel(
        functools.partial(_k2_body, n_edges),
        out_type=jax.ShapeDtypeStruct((n_edges, D), jnp.float32),
        mesh=mesh,
        compiler_params=pltpu.CompilerParams(needs_layout_passes=False),
        scratch_types=[
            pltpu.VMEM((N_NODES,), jnp.float32),
            pltpu.VMEM((N_NODES,), jnp.float32),
            pltpu.VMEM((N_NODES,), jnp.float32),
            pltpu.VMEM((D,), jnp.float32),
            pltpu.VMEM((2, K2CH), jnp.int32),
            pltpu.VMEM((2, K2CH), jnp.int32),
            pltpu.VMEM((2, K2CH, D), jnp.float32),
            pltpu.VMEM((2, K2CH, D), jnp.float32),
            pltpu.VMEM((K2CH, D), jnp.float32),
            pltpu.VMEM((K2CH,), jnp.float32),
            pltpu.SemaphoreType.DMA,
            pltpu.SemaphoreType.DMA,
            pltpu.SemaphoreType.DMA,
            pltpu.SemaphoreType.DMA,
            pltpu.SemaphoreType.DMA,
        ],
    )
    return f(a, b, row1, col1, cx_a, cy_a, cz_a, w1r)


def _k3_body(pre_ref, w2_ref, be2_ref, half_ref, full_ref):
    x = _leaky(pre_ref[...])
    y = _leaky(
        jnp.dot(x, w2_ref[...], preferred_element_type=jnp.float32)
        + be2_ref[...]
    )
    half_ref[...] = y
    full_ref[...] = y


def _k3_body_alias(pre_ref, w2_ref, be2_ref, prev_ref, half_ref, full_ref):
    del prev_ref
    _k3_body(pre_ref, w2_ref, be2_ref, half_ref, full_ref)


def _k3(pre, w2_t, be2, full_prev, s):
    blk = 2000
    n_edges = pre.shape[0]
    grid = (n_edges // blk,)
    off = s * (E_SPLIT // blk)
    in_specs = [
        pl.BlockSpec((blk, D), lambda i: (i, 0)),
        pl.BlockSpec((D, D), lambda i: (0, 0)),
        pl.BlockSpec((1, D), lambda i: (0, 0)),
    ]
    args = [pre, w2_t, be2]
    if full_prev is None:
        body, aliases = _k3_body, {}
    else:
        body, aliases = _k3_body_alias, {3: 1}
        in_specs.append(pl.BlockSpec(memory_space=pl.ANY))
        args.append(full_prev)
    return pl.pallas_call(
        body,
        grid=grid,
        in_specs=in_specs,
        out_specs=[
            pl.BlockSpec((blk, D), lambda i: (i, 0)),
            pl.BlockSpec((blk, D), lambda i: (off + i, 0)),
        ],
        out_shape=[
            jax.ShapeDtypeStruct((n_edges, D), jnp.float32),
            jax.ShapeDtypeStruct((N_EDGES, D), jnp.float32),
        ],
        input_output_aliases=aliases,
    )(*args)


N_PAD = 10240
ZROWS = 64
ROWS_PER_TILE = N_PAD // NS


def _k4_body(n_edges, feat_hbm, row_hbm, pagg_hbm, agg_sh, fbuf, idx_r, sidx,
             zb, sem_i0, sem_i1, sem_f0, sem_f1, sem_s0, sem_s1):
    NCHUNK = n_edges // CHUNK
    NI = (NCHUNK + NW - 1) // NW // 2 + 1
    cid = lax.axis_index("c")
    sid = lax.axis_index("s")
    wid = sid * NC + cid

    def zrow(i, carry):
        for j in range(D // LANES):
            zb[i, pl.ds(j * LANES, LANES)] = jnp.zeros((LANES,), jnp.float32)
        return carry

    lax.fori_loop(0, ZROWS, zrow, 0)
    for k in range(ROWS_PER_TILE // ZROWS):
        pltpu.sync_copy(
            zb, agg_sh.at[pl.ds(sid * ROWS_PER_TILE + k * ZROWS, ZROWS)])
    plsc.subcore_barrier()

    nq = (NCHUNK - wid + NW - 1) // NW
    sem_i = (sem_i0, sem_i1)
    sem_f = (sem_f0, sem_f1)
    sem_s = (sem_s0, sem_s1)

    def base_of(j):
        return (wid + jnp.minimum(j, nq - 1) * NW) * CHUNK

    def issue_idx(j, p):
        pltpu.async_copy(row_hbm.at[pl.ds(base_of(j), CHUNK)],
                         idx_r.at[p], sem_i[p])

    def wait_idx(p):
        pltpu.make_async_copy(row_hbm.at[pl.ds(0, CHUNK)],
                              idx_r.at[p], sem_i[p]).wait()

    def issue_feat(j, p):
        pltpu.async_copy(feat_hbm.at[pl.ds(base_of(j), CHUNK)],
                         fbuf.at[p], sem_f[p])

    def wait_feat(p):
        pltpu.make_async_copy(feat_hbm.at[pl.ds(0, CHUNK)],
                              fbuf.at[p], sem_f[p]).wait()

    def wait_scat(p):
        pltpu.make_async_copy(fbuf.at[p], agg_sh.at[sidx.at[p]],
                              sem_s[p]).wait()

    issue_idx(0, 0)
    issue_idx(1, 1)
    issue_feat(0, 0)

    def step(j, p):
        live = j < nq
        wait_idx(p)

        @pl.when(live)
        def _():
            for g in range(CHUNK // LANES):
                sl = pl.ds(g * LANES, LANES)
                sidx[p, sl] = idx_r[p, sl]

        issue_idx(j + 2, p)
        wait_feat(p)

        @pl.when(jnp.logical_and(j > 0, (j - 1) < nq))
        def _():
            wait_scat(1 - p)

        issue_feat(j + 1, 1 - p)

        @pl.when(live)
        def _():
            pltpu.async_copy(fbuf.at[p], agg_sh.at[sidx.at[p]], sem_s[p],
                             add=True)

    def loop_body(i, carry):
        step(2 * i, 0)
        step(2 * i + 1, 1)
        return carry

    lax.fori_loop(0, NI, loop_body, 0)

    wait_idx(0)
    wait_idx(1)
    wait_feat(0)
    plsc.subcore_barrier()

    pltpu.sync_copy(
        agg_sh.at[pl.ds(sid * ROWS_PER_TILE, ROWS_PER_TILE)],
        pagg_hbm.at[cid, pl.ds(sid * ROWS_PER_TILE, ROWS_PER_TILE)])


def _k4(feat, row1):
    n_edges = row1.shape[0]
    mesh = plsc.VectorSubcoreMesh(core_axis_name="c", subcore_axis_name="s")
    f = pl.kernel(
        functools.partial(_k4_body, n_edges),
        out_type=jax.ShapeDtypeStruct((NC, N_PAD, D), jnp.float32),
        mesh=mesh,
        compiler_params=pltpu.CompilerParams(needs_layout_passes=False),
        scratch_types=[
            pltpu.VMEM_SHARED((N_PAD, D), jnp.float32),
            pltpu.VMEM((2, CHUNK, D), jnp.float32),
            pltpu.VMEM((2, 128), jnp.int32),
            pltpu.VMEM((2, 128), jnp.int32),
            pltpu.VMEM((ZROWS, D), jnp.float32),
            pltpu.SemaphoreType.DMA,
            pltpu.SemaphoreType.DMA,
            pltpu.SemaphoreType.DMA,
            pltpu.SemaphoreType.DMA,
            pltpu.SemaphoreType.DMA,
            pltpu.SemaphoreType.DMA,
        ],
    )
    return f(feat, row1)


def _k5_body(h_ref, *rest):
    (*pagg_refs, wn1a_ref, wn1b_ref, bn1_ref, wn2_ref, bn2_ref,
     out_ref) = rest
    hb = h_ref[...]
    agg = pagg_refs[0][0] + pagg_refs[0][1]
    for pr in pagg_refs[1:]:
        agg = agg + pr[0] + pr[1]
    t = _leaky(
        jnp.dot(hb, wn1a_ref[...], preferred_element_type=jnp.float32)
        + jnp.dot(agg, wn1b_ref[...], preferred_element_type=jnp.float32)
        + bn1_ref[...]
    )
    out_ref[...] = (
        hb
        + jnp.dot(t, wn2_ref[...], preferred_element_type=jnp.float32)
        + bn2_ref[...]
    )


def _k5(h, paggs, wn1a_t, wn1b_t, bn1, wn2_t, bn2):
    blk = 2000
    grid = (N_NODES // blk,)
    return pl.pallas_call(
        _k5_body,
        grid=grid,
        in_specs=[
            pl.BlockSpec((blk, D), lambda i: (i, 0)),
            *[pl.BlockSpec((NC, blk, D), lambda i: (0, i, 0))
              for _ in paggs],
            pl.BlockSpec((D, D), lambda i: (0, 0)),
            pl.BlockSpec((D, D), lambda i: (0, 0)),
            pl.BlockSpec((1, D), lambda i: (0, 0)),
            pl.BlockSpec((D, D), lambda i: (0, 0)),
            pl.BlockSpec((1, D), lambda i: (0, 0)),
        ],
        out_specs=pl.BlockSpec((blk, D), lambda i: (i, 0)),
        out_shape=jax.ShapeDtypeStruct((N_NODES, D), jnp.float32),
    )(h, *paggs, wn1a_t, wn1b_t, bn1, wn2_t, bn2)


def kernel(h, edge_index, coord, We1, be1, We2, be2, Wn1, bn1, Wn2, bn2):
    row = edge_index[0].astype(jnp.int32)
    col = edge_index[1].astype(jnp.int32)
    coord_t = coord.T
    cx_a, cy_a, cz_a = coord_t[0], coord_t[1], coord_t[2]

    w1a_t = We1[:, :D].T
    w1b_t = We1[:, D:2 * D].T
    w1r = We1[:, 2 * D]
    be1_2 = be1.reshape(1, D)
    w2_t = We2.T
    be2_2 = be2.reshape(1, D)
    wn1a_t = Wn1[:, :D].T
    wn1b_t = Wn1[:, D:].T
    bn1_2 = bn1.reshape(1, D)
    wn2_t = Wn2.T
    bn2_2 = bn2.reshape(1, D)

    a, b = _k1(h, w1a_t, w1b_t, be1_2)
    paggs = []
    full = None
    for s in range(NSPLIT):
        rs = row[s * E_SPLIT:(s + 1) * E_SPLIT]
        cs = col[s * E_SPLIT:(s + 1) * E_SPLIT]
        pre = _k2(a, b, rs, cs, cx_a, cy_a, cz_a, w1r)
        feat, full = _k3(pre, w2_t, be2_2, full, s)
        paggs.append(_k4(feat, rs))
    h_out = _k5(h, paggs, wn1a_t, wn1b_t, bn1_2, wn2_t, bn2_2)
    return (h_out, coord, full)


# speedup vs baseline: 1.1128x; 1.1128x over previous
import functools

import jax
import jax.numpy as jnp
from jax import lax
from jax.experimental import pallas as pl
from jax.experimental.pallas import tpu as pltpu
from jax.experimental.pallas import tpu_sc as plsc

N_NODES = 10000
N_EDGES = 320000
D = 128

NC = 2
NS = 16
NW = NC * NS
LANES = 16

CHUNK = 128
NSPLIT = 2
E_SPLIT = N_EDGES // NSPLIT


def _leaky(x):
    return jnp.where(x > 0, x, 0.2 * x)


def _k1_body(h_ref, w1a_ref, w1b_ref, be1_ref, a_ref, b_ref):
    hb = h_ref[...]
    a_ref[...] = (
        jnp.dot(hb, w1a_ref[...], preferred_element_type=jnp.float32)
        + be1_ref[...]
    )
    b_ref[...] = jnp.dot(hb, w1b_ref[...], preferred_element_type=jnp.float32)


def _k1(h, w1a_t, w1b_t, be1):
    blk = 2000
    grid = (N_NODES // blk,)
    return pl.pallas_call(
        _k1_body,
        grid=grid,
        in_specs=[
            pl.BlockSpec((blk, D), lambda i: (i, 0)),
            pl.BlockSpec((D, D), lambda i: (0, 0)),
            pl.BlockSpec((D, D), lambda i: (0, 0)),
            pl.BlockSpec((1, D), lambda i: (0, 0)),
        ],
        out_specs=[
            pl.BlockSpec((blk, D), lambda i: (i, 0)),
            pl.BlockSpec((blk, D), lambda i: (i, 0)),
        ],
        out_shape=[
            jax.ShapeDtypeStruct((N_NODES, D), jnp.float32),
            jax.ShapeDtypeStruct((N_NODES, D), jnp.float32),
        ],
    )(h, w1a_t, w1b_t, be1)


K2CH = 128


def _k2_body(n_edges, a_hbm, b_hbm, row_hbm, col_hbm, cx_hbm, cy_hbm, cz_hbm,
             w1r_hbm, pre_hbm,
             cx, cy, cz, w1r_v, idx_r, idx_c, arow, bcol, obuf, radial,
             sem_i0, sem_i1, sem_g0, sem_g1, sem_out):
    NCHUNK = n_edges // K2CH
    NI = (NCHUNK + NW - 1) // NW // 2 + 1
    cid = lax.axis_index("c")
    sid = lax.axis_index("s")
    wid = sid * NC + cid

    pltpu.sync_copy(cx_hbm, cx)
    pltpu.sync_copy(cy_hbm, cy)
    pltpu.sync_copy(cz_hbm, cz)
    pltpu.sync_copy(w1r_hbm, w1r_v)
    w1rj = [w1r_v[pl.ds(j * LANES, LANES)] for j in range(D // LANES)]

    nq = (NCHUNK - wid + NW - 1) // NW
    sem_i = (sem_i0, sem_i1)
    sem_g = (sem_g0, sem_g1)

    def base_of(j):
        return (wid + jnp.minimum(j, nq - 1) * NW) * K2CH

    def issue_idx(j, p):
        pltpu.async_copy(row_hbm.at[pl.ds(base_of(j), K2CH)],
                         idx_r.at[p], sem_i[p])
        pltpu.async_copy(col_hbm.at[pl.ds(base_of(j), K2CH)],
                         idx_c.at[p], sem_i[p])

    def wait_idx(p):
        pltpu.make_async_copy(row_hbm.at[pl.ds(0, K2CH)],
                              idx_r.at[p], sem_i[p]).wait()
        pltpu.make_async_copy(col_hbm.at[pl.ds(0, K2CH)],
                              idx_c.at[p], sem_i[p]).wait()

    def issue_gat(p):
        pltpu.async_copy(a_hbm.at[idx_r.at[p]], arow.at[p], sem_g[p])
        pltpu.async_copy(b_hbm.at[idx_c.at[p]], bcol.at[p], sem_g[p])

    def wait_gat(p):
        pltpu.make_async_copy(a_hbm.at[idx_r.at[p]], arow.at[p],
                              sem_g[p]).wait()
        pltpu.make_async_copy(b_hbm.at[idx_c.at[p]], bcol.at[p],
                              sem_g[p]).wait()

    def wait_out():
        pltpu.make_async_copy(obuf, pre_hbm.at[pl.ds(0, K2CH)],
                              sem_out).wait()

    issue_idx(0, 0)
    issue_idx(1, 1)
    wait_idx(0)
    issue_gat(0)

    def step(j, p):
        live = j < nq
        wait_gat(p)

        @pl.when(live)
        def _():
            for g in range(K2CH // LANES):
                r16 = idx_r[p, pl.ds(g * LANES, LANES)]
                c16 = idx_c[p, pl.ds(g * LANES, LANES)]
                dx = plsc.load_gather(cx, [r16]) - plsc.load_gather(cx, [c16])
                dy = plsc.load_gather(cy, [r16]) - plsc.load_gather(cy, [c16])
                dz = plsc.load_gather(cz, [r16]) - plsc.load_gather(cz, [c16])
                radial[pl.ds(g * LANES, LANES)] = dx * dx + dy * dy + dz * dz

        issue_idx(j + 2, p)
        wait_idx(1 - p)
        issue_gat(1 - p)

        @pl.when(jnp.logical_and(j > 0, live))
        def _():
            wait_out()

        @pl.when(live)
        def _():
            arow_p = arow.at[p]
            bcol_p = bcol.at[p]

            def edge_body(e, carry2):
                rb = plsc.load_gather(
                    radial, [jnp.full((LANES,), e, jnp.int32)])
                for j2 in range(D // LANES):
                    sl = pl.ds(j2 * LANES, LANES)
                    obuf[e, sl] = arow_p[e, sl] + bcol_p[e, sl] + rb * w1rj[j2]
                return carry2

            lax.fori_loop(0, K2CH, edge_body, 0, unroll=4)
            pltpu.async_copy(obuf, pre_hbm.at[pl.ds(base_of(j), K2CH)],
                             sem_out)

    def loop_body(i, carry):
        step(2 * i, 0)
        step(2 * i + 1, 1)
        return carry

    lax.fori_loop(0, NI, loop_body, 0)

    wait_out()
    wait_gat(0)
    wait_idx(1)


def _k2(a, b, row1, col1, cx_a, cy_a, cz_a, w1r):
    n_edges = row1.shape[0]
    mesh = plsc.VectorSubcoreMesh(core_axis_name="c", subcore_axis_name="s")
    f = pl.kernel(
        functools.partial(_k2_body, n_edges),
        out_type=jax.ShapeDtypeStruct((n_edges, D), jnp.float32),
        mesh=mesh,
        compiler_params=pltpu.CompilerParams(needs_layout_passes=False),
        scratch_types=[
            pltpu.VMEM((N_NODES,), jnp.float32),
            pltpu.VMEM((N_NODES,), jnp.float32),
            pltpu.VMEM((N_NODES,), jnp.float32),
            pltpu.VMEM((D,), jnp.float32),
            pltpu.VMEM((2, K2CH), jnp.int32),
            pltpu.VMEM((2, K2CH), jnp.int32),
            pltpu.VMEM((2, K2CH, D), jnp.float32),
            pltpu.VMEM((2, K2CH, D), jnp.float32),
            pltpu.VMEM((K2CH, D), jnp.float32),
            pltpu.VMEM((K2CH,), jnp.float32),
            pltpu.SemaphoreType.DMA,
            pltpu.SemaphoreType.DMA,
            pltpu.SemaphoreType.DMA,
            pltpu.SemaphoreType.DMA,
            pltpu.SemaphoreType.DMA,
        ],
    )
    return f(a, b, row1, col1, cx_a, cy_a, cz_a, w1r)


def _k3_body(pre_ref, w2_ref, be2_ref, half_ref, full_ref):
    x = _leaky(pre_ref[...])
    y = _leaky(
        jnp.dot(x, w2_ref[...], preferred_element_type=jnp.float32)
        + be2_ref[...]
    )
    half_ref[...] = y
    full_ref[...] = y


def _k3_body_alias(pre_ref, w2_ref, be2_ref, prev_ref, half_ref, full_ref):
    del prev_ref
    _k3_body(pre_ref, w2_ref, be2_ref, half_ref, full_ref)


def _k3(pre, w2_t, be2, full_prev, s):
    blk = 2000
    n_edges = pre.shape[0]
    grid = (n_edges // blk,)
    off = s * (E_SPLIT // blk)
    in_specs = [
        pl.BlockSpec((blk, D), lambda i: (i, 0)),
        pl.BlockSpec((D, D), lambda i: (0, 0)),
        pl.BlockSpec((1, D), lambda i: (0, 0)),
    ]
    args = [pre, w2_t, be2]
    if full_prev is None:
        body, aliases = _k3_body, {}
    else:
        body, aliases = _k3_body_alias, {3: 1}
        in_specs.append(pl.BlockSpec(memory_space=pl.ANY))
        args.append(full_prev)
    return pl.pallas_call(
        body,
        grid=grid,
        in_specs=in_specs,
        out_specs=[
            pl.BlockSpec((blk, D), lambda i: (i, 0)),
            pl.BlockSpec((blk, D), lambda i: (off + i, 0)),
        ],
        out_shape=[
            jax.ShapeDtypeStruct((n_edges, D), jnp.float32),
            jax.ShapeDtypeStruct((N_EDGES, D), jnp.float32),
        ],
        input_output_aliases=aliases,
    )(*args)


N_PAD = 10240
ZROWS = 64
ROWS_PER_TILE = N_PAD // NS


def _k4_body(n_edges, feat_hbm, row_hbm, pagg_hbm, agg_sh, fbuf, idx_r, sidx,
             zb, sem_i0, sem_i1, sem_f0, sem_f1, sem_s0, sem_s1):
    NCHUNK = n_edges // CHUNK
    NI = (NCHUNK + NW - 1) // NW // 2 + 1
    cid = lax.axis_index("c")
    sid = lax.axis_index("s")
    wid = sid * NC + cid

    def zrow(i, carry):
        for j in range(D // LANES):
            zb[i, pl.ds(j * LANES, LANES)] = jnp.zeros((LANES,), jnp.float32)
        return carry

    lax.fori_loop(0, ZROWS, zrow, 0)
    for k in range(ROWS_PER_TILE // ZROWS):
        pltpu.sync_copy(
            zb, agg_sh.at[pl.ds(sid * ROWS_PER_TILE + k * ZROWS, ZROWS)])
    plsc.subcore_barrier()

    nq = (NCHUNK - wid + NW - 1) // NW
    sem_i = (sem_i0, sem_i1)
    sem_f = (sem_f0, sem_f1)
    sem_s = (sem_s0, sem_s1)

    def base_of(j):
        return (wid + jnp.minimum(j, nq - 1) * NW) * CHUNK

    def issue_idx(j, p):
        pltpu.async_copy(row_hbm.at[pl.ds(base_of(j), CHUNK)],
                         idx_r.at[p], sem_i[p])

    def wait_idx(p):
        pltpu.make_async_copy(row_hbm.at[pl.ds(0, CHUNK)],
                              idx_r.at[p], sem_i[p]).wait()

    def issue_feat(j, p):
        pltpu.async_copy(feat_hbm.at[pl.ds(base_of(j), CHUNK)],
                         fbuf.at[p], sem_f[p])

    def wait_feat(p):
        pltpu.make_async_copy(feat_hbm.at[pl.ds(0, CHUNK)],
                              fbuf.at[p], sem_f[p]).wait()

    def wait_scat(p):
        pltpu.make_async_copy(fbuf.at[p], agg_sh.at[sidx.at[p]],
                              sem_s[p]).wait()

    issue_idx(0, 0)
    issue_idx(1, 1)
    issue_feat(0, 0)

    def step(j, p):
        live = j < nq
        wait_idx(p)

        @pl.when(live)
        def _():
            for g in range(CHUNK // LANES):
                sl = pl.ds(g * LANES, LANES)
                sidx[p, sl] = idx_r[p, sl]

        issue_idx(j + 2, p)
        wait_feat(p)

        @pl.when(jnp.logical_and(j > 0, (j - 1) < nq))
        def _():
            wait_scat(1 - p)

        issue_feat(j + 1, 1 - p)

        @pl.when(live)
        def _():
            pltpu.async_copy(fbuf.at[p], agg_sh.at[sidx.at[p]], sem_s[p],
                             add=True)

    def loop_body(i, carry):
        step(2 * i, 0)
        step(2 * i + 1, 1)
        return carry

    lax.fori_loop(0, NI, loop_body, 0)

    wait_idx(0)
    wait_idx(1)
    wait_feat(0)
    plsc.subcore_barrier()

    pltpu.sync_copy(
        agg_sh.at[pl.ds(sid * ROWS_PER_TILE, ROWS_PER_TILE)],
        pagg_hbm.at[cid, pl.ds(sid * ROWS_PER_TILE, ROWS_PER_TILE)])


def _k4(feat, row1):
    n_edges = row1.shape[0]
    mesh = plsc.VectorSubcoreMesh(core_axis_name="c", subcore_axis_name="s")
    f = pl.kernel(
        functools.partial(_k4_body, n_edges),
        out_type=jax.ShapeDtypeStruct((NC, N_PAD, D), jnp.float32),
        mesh=mesh,
        compiler_params=pltpu.CompilerParams(needs_layout_passes=False),
        scratch_types=[
            pltpu.VMEM_SHARED((N_PAD, D), jnp.float32),
            pltpu.VMEM((2, CHUNK, D), jnp.float32),
            pltpu.VMEM((2, 128), jnp.int32),
            pltpu.VMEM((2, 128), jnp.int32),
            pltpu.VMEM((ZROWS, D), jnp.float32),
            pltpu.SemaphoreType.DMA,
            pltpu.SemaphoreType.DMA,
            pltpu.SemaphoreType.DMA,
            pltpu.SemaphoreType.DMA,
            pltpu.SemaphoreType.DMA,
            pltpu.SemaphoreType.DMA,
        ],
    )
    return f(feat, row1)


def _k5_body(h_ref, *rest):
    (*pagg_refs, wn1a_ref, wn1b_ref, bn1_ref, wn2_ref, bn2_ref,
     out_ref) = rest
    hb = h_ref[...]
    agg = pagg_refs[0][0] + pagg_refs[0][1]
    for pr in pagg_refs[1:]:
        agg = agg + pr[0] + pr[1]
    t = _leaky(
        jnp.dot(hb, wn1a_ref[...], preferred_element_type=jnp.float32)
        + jnp.dot(agg, wn1b_ref[...], preferred_element_type=jnp.float32)
        + bn1_ref[...]
    )
    out_ref[...] = (
        hb
        + jnp.dot(t, wn2_ref[...], preferred_element_type=jnp.float32)
        + bn2_ref[...]
    )


def _k5(h, paggs, wn1a_t, wn1b_t, bn1, wn2_t, bn2):
    blk = 2000
    grid = (N_NODES // blk,)
    return pl.pallas_call(
        _k5_body,
        grid=grid,
        in_specs=[
            pl.BlockSpec((blk, D), lambda i: (i, 0)),
            *[pl.BlockSpec((NC, blk, D), lambda i: (0, i, 0))
              for _ in paggs],
            pl.BlockSpec((D, D), lambda i: (0, 0)),
            pl.BlockSpec((D, D), lambda i: (0, 0)),
            pl.BlockSpec((1, D), lambda i: (0, 0)),
            pl.BlockSpec((D, D), lambda i: (0, 0)),
            pl.BlockSpec((1, D), lambda i: (0, 0)),
        ],
        out_specs=pl.BlockSpec((blk, D), lambda i: (i, 0)),
        out_shape=jax.ShapeDtypeStruct((N_NODES, D), jnp.float32),
    )(h, *paggs, wn1a_t, wn1b_t, bn1, wn2_t, bn2)


def kernel(h, edge_index, coord, We1, be1, We2, be2, Wn1, bn1, Wn2, bn2):
    row = edge_index[0].astype(jnp.int32)
    col = edge_index[1].astype(jnp.int32)
    coord_t = coord.T
    cx_a, cy_a, cz_a = coord_t[0], coord_t[1], coord_t[2]

    w1a_t = We1[:, :D].T
    w1b_t = We1[:, D:2 * D].T
    w1r = We1[:, 2 * D]
    be1_2 = be1.reshape(1, D)
    w2_t = We2.T
    be2_2 = be2.reshape(1, D)
    wn1a_t = Wn1[:, :D].T
    wn1b_t = Wn1[:, D:].T
    bn1_2 = bn1.reshape(1, D)
    wn2_t = Wn2.T
    bn2_2 = bn2.reshape(1, D)

    a, b = _k1(h, w1a_t, w1b_t, be1_2)
    paggs = []
    full = None
    for s in range(NSPLIT):
        rs = row[s * E_SPLIT:(s + 1) * E_SPLIT]
        cs = col[s * E_SPLIT:(s + 1) * E_SPLIT]
        pre = _k2(a, b, rs, cs, cx_a, cy_a, cz_a, w1r)
        feat, full = _k3(pre, w2_t, be2_2, full, s)
        paggs.append(_k4(feat, rs))
    h_out = _k5(h, paggs, wn1a_t, wn1b_t, bn1_2, wn2_t, bn2_2)
    return (h_out, coord, full)
